# Initial kernel scaffold; baseline (speedup 1.0000x reference)
import jax
import jax.numpy as jnp
from jax.experimental import pallas as pl


def kernel(input, emb_weight, lin_w, lin_b):
    raise NotImplementedError("write your pallas kernel here")


# speedup vs baseline: 20.6576x; 20.6576x over previous
import jax
import jax.numpy as jnp
from jax import lax
from jax.experimental import pallas as pl
from jax.experimental.pallas import tpu as pltpu
from jax.experimental.pallas import tpu_sc as plsc

B = 16384
L = 200
V = 1000
D_IN = 100
D_OUT = 21
DPAD = 32
NC, NS = 2, 16
NW = NC * NS
BAGS_PER_W = B // NW
CB = 8
GCHUNK = 80


def _table_body(emb_ref, w_ref, out_ref):
    out_ref[...] = jnp.dot(
        emb_ref[...], w_ref[...].T, preferred_element_type=jnp.float32
    ) * (1.0 / L)


def _fused_table(emb_weight, w_pad):
    return pl.pallas_call(
        _table_body,
        out_shape=jax.ShapeDtypeStruct((V, DPAD), jnp.float32),
    )(emb_weight, w_pad)


def _sc_body(table_hbm, idx_hbm, bias_hbm, out_hbm,
             idx_v, rows_v, acc_v, bias_v, sem):
    wid = lax.axis_index("s") * NC + lax.axis_index("c")
    pltpu.sync_copy(bias_hbm, bias_v)

    @pl.loop(0, BAGS_PER_W, step=CB)
    def _(cb):
        bag0 = wid * BAGS_PER_W + cb
        pltpu.sync_copy(idx_hbm.at[pl.ds(bag0 * L, CB * L)], idx_v)
        copies = []
        for k in range(CB * L // GCHUNK):
            copies.append(pltpu.async_copy(
                table_hbm.at[idx_v.at[pl.ds(k * GCHUNK, GCHUNK)]],
                rows_v.at[pl.ds(k * GCHUNK, GCHUNK)], sem))
        for c in copies:
            c.wait()
        for j in range(CB):
            def body(r, accs):
                a0, a1 = accs
                a0 = a0 + rows_v[j * L + r, pl.ds(0, 16)]
                a1 = a1 + rows_v[j * L + r, pl.ds(16, 16)]
                return a0, a1
            a0, a1 = lax.fori_loop(
                0, L, body, (bias_v[pl.ds(0, 16)], bias_v[pl.ds(16, 16)]))
            acc_v[j, pl.ds(0, 16)] = a0
            acc_v[j, pl.ds(16, 16)] = a1
        pltpu.sync_copy(acc_v, out_hbm.at[pl.ds(bag0, CB)])


def kernel(input, emb_weight, lin_w, lin_b):
    idx = jnp.asarray(input, jnp.int32).reshape(-1)
    w_pad = jnp.zeros((DPAD, D_IN), jnp.float32).at[:D_OUT].set(
        lin_w.astype(jnp.float32))
    b_pad = jnp.zeros((DPAD,), jnp.float32).at[:D_OUT].set(
        lin_b.astype(jnp.float32))
    table = _fused_table(emb_weight.astype(jnp.float32), w_pad)

    mesh = plsc.VectorSubcoreMesh(core_axis_name="c", subcore_axis_name="s")
    bag_sum = pl.kernel(
        _sc_body,
        mesh=mesh,
        compiler_params=pltpu.CompilerParams(use_tc_tiling_on_sc=False),
        out_type=jax.ShapeDtypeStruct((B, DPAD), jnp.float32),
        scratch_types=[
            pltpu.VMEM((CB * L,), jnp.int32),
            pltpu.VMEM((CB * L, DPAD), jnp.float32),
            pltpu.VMEM((CB, DPAD), jnp.float32),
            pltpu.VMEM((DPAD,), jnp.float32),
            pltpu.SemaphoreType.DMA,
        ],
    )
    out = bag_sum(table, idx, b_pad)
    return out[:, :D_OUT]


# speedup vs baseline: 24.3471x; 1.1786x over previous
import jax
import jax.numpy as jnp
from jax import lax
from jax.experimental import pallas as pl
from jax.experimental.pallas import tpu as pltpu
from jax.experimental.pallas import tpu_sc as plsc

B = 16384
L = 200
V = 1000
D_IN = 100
D_OUT = 21
DPAD = 32
NC, NS = 2, 16
NW = NC * NS
BAGS_PER_W = B // NW
CB = 8
GCHUNK = 80
UN = 25
NACC = 4


def _table_body(emb_ref, w_ref, out_ref):
    out_ref[...] = jnp.dot(
        emb_ref[...], w_ref[...].T, preferred_element_type=jnp.float32
    ) * (1.0 / L)


def _fused_table(emb_weight, w_pad):
    return pl.pallas_call(
        _table_body,
        out_shape=jax.ShapeDtypeStruct((V, DPAD), jnp.float32),
    )(emb_weight, w_pad)


def _sc_body(table_hbm, idx_hbm, bias_hbm, out_hbm,
             idx_v, rows_v, acc_v, bias_v, sem):
    wid = lax.axis_index("s") * NC + lax.axis_index("c")
    pltpu.sync_copy(bias_hbm, bias_v)

    @pl.loop(0, BAGS_PER_W, step=CB)
    def _(cb):
        bag0 = wid * BAGS_PER_W + cb
        pltpu.sync_copy(idx_hbm.at[pl.ds(bag0 * L, CB * L)], idx_v)
        copies = []
        for k in range(CB * L // GCHUNK):
            copies.append(pltpu.async_copy(
                table_hbm.at[idx_v.at[pl.ds(k * GCHUNK, GCHUNK)]],
                rows_v.at[pl.ds(k * GCHUNK, GCHUNK)], sem))
        for c in copies:
            c.wait()

        @pl.loop(0, CB)
        def _(j):
            base_row = j * L

            def body(i, accs):
                a = list(accs)
                r0 = base_row + i * UN
                for u in range(UN):
                    p = u % NACC
                    a[2 * p] = a[2 * p] + rows_v[r0 + u, pl.ds(0, 16)]
                    a[2 * p + 1] = a[2 * p + 1] + rows_v[r0 + u, pl.ds(16, 16)]
                return tuple(a)

            zero = jnp.zeros((16,), jnp.float32)
            init = [zero] * (2 * NACC)
            init[0] = bias_v[pl.ds(0, 16)]
            init[1] = bias_v[pl.ds(16, 16)]
            accs = lax.fori_loop(0, L // UN, body, tuple(init))
            a0 = accs[0]
            a1 = accs[1]
            for p in range(1, NACC):
                a0 = a0 + accs[2 * p]
                a1 = a1 + accs[2 * p + 1]
            acc_v[j, pl.ds(0, 16)] = a0
            acc_v[j, pl.ds(16, 16)] = a1

        pltpu.sync_copy(acc_v, out_hbm.at[pl.ds(bag0, CB)])


def kernel(input, emb_weight, lin_w, lin_b):
    idx = jnp.asarray(input, jnp.int32).reshape(-1)
    w_pad = jnp.zeros((DPAD, D_IN), jnp.float32).at[:D_OUT].set(
        lin_w.astype(jnp.float32))
    b_pad = jnp.zeros((DPAD,), jnp.float32).at[:D_OUT].set(
        lin_b.astype(jnp.float32))
    table = _fused_table(emb_weight.astype(jnp.float32), w_pad)

    mesh = plsc.VectorSubcoreMesh(core_axis_name="c", subcore_axis_name="s")
    bag_sum = pl.kernel(
        _sc_body,
        mesh=mesh,
        compiler_params=pltpu.CompilerParams(use_tc_tiling_on_sc=False),
        out_type=jax.ShapeDtypeStruct((B, DPAD), jnp.float32),
        scratch_types=[
            pltpu.VMEM((CB * L,), jnp.int32),
            pltpu.VMEM((CB * L, DPAD), jnp.float32),
            pltpu.VMEM((CB, DPAD), jnp.float32),
            pltpu.VMEM((DPAD,), jnp.float32),
            pltpu.SemaphoreType.DMA,
        ],
    )
    out = bag_sum(table, idx, b_pad)
    return out[:, :D_OUT]


# speedup vs baseline: 62.1698x; 2.5535x over previous
import jax
import jax.numpy as jnp
from jax import lax
from jax.experimental import pallas as pl
from jax.experimental.pallas import tpu as pltpu
from jax.experimental.pallas import tpu_sc as plsc

B = 16384
L = 200
V = 1000
D_IN = 100
D_OUT = 21
DPAD = 24
VPAD = 1008
NC, NS = 2, 16
NW = NC * NS
BAGS_PER_W = B // NW
NG = BAGS_PER_W // 16
IDX_PER_W = BAGS_PER_W * L


def _table_body(emb_ref, w_ref, b_ref, out_ref):
    p = jnp.dot(emb_ref[...], w_ref[...].T,
                preferred_element_type=jnp.float32) * (1.0 / L)
    out_ref[...] = jnp.concatenate([p, b_ref[...]], axis=0)


def _fused_table(emb_weight, w_pad, b_rows):
    return pl.pallas_call(
        _table_body,
        out_shape=jax.ShapeDtypeStruct((VPAD, DPAD), jnp.float32),
    )(emb_weight, w_pad, b_rows)


def _sc_body(table_hbm, idx_hbm, out_hbm, table_v, idx_v, ob0, ob1,
             sem, osem0, osem1):
    wid = lax.axis_index("s") * NC + lax.axis_index("c")
    pltpu.sync_copy(table_hbm, table_v)
    pltpu.sync_copy(idx_hbm.at[pl.ds(wid * IDX_PER_W, IDX_PER_W)],
                    idx_v.at[pl.ds(0, IDX_PER_W)])

    lanes = lax.iota(jnp.int32, 16)
    lane_off = lanes * L
    bias_row = jnp.full((16,), V, jnp.int32)
    cols = [jnp.full((16,), c, jnp.int32) for c in range(D_OUT)]

    def do_group(g, ob, osem):
        gbase = g * (16 * L)
        vidx0 = plsc.load_gather(idx_v, [lane_off + gbase])

        def body(l, carry):
            vidx = carry[0]
            a = list(carry[1:])
            vidx_next = plsc.load_gather(idx_v, [lane_off + (gbase + l + 1)])
            for c in range(D_OUT):
                a[c] = a[c] + plsc.load_gather(table_v, [vidx, cols[c]])
            return (vidx_next,) + tuple(a)

        init = tuple(plsc.load_gather(table_v, [bias_row, cols[c]])
                     for c in range(D_OUT))
        accs = lax.fori_loop(0, L, body, (vidx0,) + init)[1:]
        for c in range(D_OUT):
            plsc.store_scatter(ob, [lanes, cols[c]], accs[c])
        return pltpu.async_copy(
            ob, out_hbm.at[pl.ds(wid * BAGS_PER_W + g * 16, 16)], osem)

    @pl.loop(0, NG, step=2)
    def _(g):
        h0 = do_group(g, ob0, osem0)
        h1 = do_group(g + 1, ob1, osem1)
        h0.wait()
        h1.wait()


def kernel(input, emb_weight, lin_w, lin_b):
    idx = jnp.asarray(input, jnp.int32).reshape(-1)
    w_pad = jnp.zeros((DPAD, D_IN), jnp.float32).at[:D_OUT].set(
        lin_w.astype(jnp.float32))
    b_rows = jnp.zeros((VPAD - V, DPAD), jnp.float32).at[:, :D_OUT].set(
        lin_b.astype(jnp.float32))
    table = _fused_table(emb_weight.astype(jnp.float32), w_pad, b_rows)

    mesh = plsc.VectorSubcoreMesh(core_axis_name="c", subcore_axis_name="s")
    bag_sum = pl.kernel(
        _sc_body,
        mesh=mesh,
        compiler_params=pltpu.CompilerParams(
            use_tc_tiling_on_sc=False, needs_layout_passes=False),
        out_type=jax.ShapeDtypeStruct((B, DPAD), jnp.float32),
        scratch_types=[
            pltpu.VMEM((VPAD, DPAD), jnp.float32),
            pltpu.VMEM((IDX_PER_W + 16,), jnp.int32),
            pltpu.VMEM((16, DPAD), jnp.float32),
            pltpu.VMEM((16, DPAD), jnp.float32),
            pltpu.SemaphoreType.DMA,
            pltpu.SemaphoreType.DMA,
            pltpu.SemaphoreType.DMA,
        ],
    )
    out = bag_sum(table, idx)
    return out[:, :D_OUT]
